# baseline (speedup 1.0000x reference)
import functools

import jax
import jax.numpy as jnp
from jax import lax
from jax.experimental import pallas as pl
from jax.experimental.pallas import tpu as pltpu
from jax.experimental.pallas import tpu_sc as plsc

_SPARSE_RATIO = 0.5
_NC, _NS, _LANES = 2, 16, 16
_NW = _NC * _NS


def _wid():
    return lax.axis_index("s") * _NC + lax.axis_index("c")


def _mesh():
    return plsc.VectorSubcoreMesh(core_axis_name="c", subcore_axis_name="s")


def _select_body(K, score_hbm, idx_hbm, sbits_v, idxbuf_v):
    L = score_hbm.shape[0]
    nv = L // _LANES
    w = _wid()

    @pl.when(w == 0)
    def _():
        pltpu.sync_copy(score_hbm, sbits_v)
        lane = lax.iota(jnp.int32, _LANES)

        def count_ge(t):
            def body(i, acc):
                v = sbits_v[pl.ds(i * _LANES, _LANES)]
                return acc + jnp.where(v >= t, 1, 0)

            acc = lax.fori_loop(0, nv, body, jnp.zeros((_LANES,), jnp.int32),
                                unroll=8)
            return jnp.sum(acc)

        def step(k, prefix):
            cand = prefix | (jnp.int32(1) << (30 - k))
            return jnp.where(count_ge(cand) >= K, cand, prefix)

        thr = lax.fori_loop(0, 31, step, jnp.int32(0))
        n_eq = K - count_ge(thr + 1)

        def emit(i, carry):
            off, eq_seen = carry
            v = sbits_v[pl.ds(i * _LANES, _LANES)]
            gt = v > thr
            eq = v == thr
            eqi = jnp.where(eq, 1, 0)
            excl = plsc.cumsum(eqi) - eqi
            keep = gt | (eq & ((eq_seen + excl) < n_eq))
            plsc.store_compressed(idxbuf_v.at[pl.ds(off, _LANES)],
                                  lane + i * _LANES, mask=keep)
            return (off + jnp.sum(jnp.where(keep, 1, 0)),
                    eq_seen + jnp.sum(eqi))

        lax.fori_loop(0, nv, emit, (jnp.int32(0), jnp.int32(0)), unroll=4)
        pltpu.sync_copy(idxbuf_v.at[pl.ds(0, K)], idx_hbm)


def _sc_select1(sbits, K):
    (L,) = sbits.shape
    return pl.kernel(
        functools.partial(_select_body, K),
        out_type=jax.ShapeDtypeStruct((K,), jnp.int32),
        mesh=_mesh(),
        scratch_types=[
            pltpu.VMEM((L,), jnp.int32),
            pltpu.VMEM((K + _LANES,), jnp.int32),
        ],
        compiler_params=pltpu.CompilerParams(needs_layout_passes=False),
    )(sbits)


_CH = 16


def _gather_body(b, x_hbm, idx_hbm, out_hbm, idx_v, buf0, buf1, buf2,
                 g0, g1, g2, s0, s1, s2):
    L, C = x_hbm.shape
    K = idx_hbm.shape[0]
    rows = K // _NW
    nch = rows // _CH
    w = _wid()
    base = w * rows

    pltpu.sync_copy(idx_hbm.at[pl.ds(base, rows)], idx_v)

    bufs = (buf0, buf1, buf2)
    gsems = (g0, g1, g2)
    ssems = (s0, s1, s2)

    def gth(j, t):
        return pltpu.make_async_copy(
            x_hbm.at[idx_v.at[pl.ds(j * _CH, _CH)]], bufs[t], gsems[t])

    def sct(j, t):
        return pltpu.make_async_copy(
            bufs[t], out_hbm.at[b, pl.ds(base + j * _CH, _CH), :], ssems[t])

    gth(0, 0).start()
    gth(1, 1).start()
    gth(0, 0).wait()
    sct(0, 0).start()
    gth(2, 2).start()
    gth(1, 1).wait()
    sct(1, 1).start()
    sct(0, 0).wait()
    gth(3, 0).start()

    def three(k, _):
        j0 = 2 + k * 3
        for t_off in range(3):
            j = j0 + t_off
            t = (2 + t_off) % 3
            gth(j, t).wait()
            sct(j, t).start()
            sct(j - 1, (t + 2) % 3).wait()
            gth(j + 2, (t + 2) % 3).start()
        return ()

    lax.fori_loop(0, (nch - 5) // 3, three, ())
    j = nch - 3
    gth(j, j % 3).wait()
    sct(j, j % 3).start()
    sct(j - 1, (j - 1) % 3).wait()
    gth(j + 2, (j + 2) % 3).start()
    j = nch - 2
    gth(j, j % 3).wait()
    sct(j, j % 3).start()
    sct(j - 1, (j - 1) % 3).wait()
    j = nch - 1
    gth(j, j % 3).wait()
    sct(j, j % 3).start()
    sct(j - 1, (j - 1) % 3).wait()
    sct(j, j % 3).wait()


def _sc_gather1(b, xb, idx, out_ref):
    L, C = xb.shape
    K = idx.shape[0]
    rows = K // _NW
    pl.kernel(
        functools.partial(_gather_body, b),
        out_type=(),
        mesh=_mesh(),
        scratch_types=[
            pltpu.VMEM((rows,), jnp.int32),
            pltpu.VMEM((_CH, C), jnp.float32),
            pltpu.VMEM((_CH, C), jnp.float32),
            pltpu.VMEM((_CH, C), jnp.float32),
            pltpu.SemaphoreType.DMA,
            pltpu.SemaphoreType.DMA,
            pltpu.SemaphoreType.DMA,
            pltpu.SemaphoreType.DMA,
            pltpu.SemaphoreType.DMA,
            pltpu.SemaphoreType.DMA,
        ],
        compiler_params=pltpu.CompilerParams(needs_layout_passes=False),
    )(xb, idx, out_ref)


def kernel(x):
    B, L, C = x.shape
    K = max(1, int(L * (1.0 - _SPARSE_RATIO)))
    out_ref = jax.new_ref(jnp.zeros((B, K, C), jnp.float32))
    idxs = []
    for b in range(B):
        xb = lax.index_in_dim(x, b, 0, keepdims=False)
        score = jnp.sqrt(jnp.sum(xb * xb, axis=-1))
        idx_b = _sc_select1(lax.bitcast_convert_type(score, jnp.int32), K)
        _sc_gather1(b, xb, idx_b, out_ref)
        idxs.append(idx_b)
    indices = jnp.stack(idxs, axis=0)
    return (out_ref[...], indices)


# speedup vs baseline: 2.1216x; 2.1216x over previous
import functools

import jax
import jax.numpy as jnp
from jax import lax
from jax.experimental import pallas as pl
from jax.experimental.pallas import tpu as pltpu
from jax.experimental.pallas import tpu_sc as plsc

_SPARSE_RATIO = 0.5
_NC, _NS, _LANES = 2, 16, 16
_NW = _NC * _NS


def _wid():
    return lax.axis_index("s") * _NC + lax.axis_index("c")


_NB = 256


def _select_body(K, score_hbm, idx_hbm, sbits_v, idxbuf_v, hist_v, cand_v):
    B, L = score_hbm.shape
    nv = L // _LANES
    w = _wid()

    @pl.when(w < B)
    def _():
        pltpu.sync_copy(score_hbm.at[w], sbits_v)
        lane = lax.iota(jnp.int32, _LANES)

        def mm(i, c):
            v = sbits_v[pl.ds(i * _LANES, _LANES)]
            return (jnp.minimum(c[0], v), jnp.maximum(c[1], v))

        mnv, mxv = lax.fori_loop(
            0, nv, mm,
            (jnp.full((_LANES,), jnp.int32(0x7F800000)),
             jnp.zeros((_LANES,), jnp.int32)), unroll=8)
        mn = jnp.min(mnv)
        mx = jnp.max(mxv)
        rng = mx - mn

        def shstep(s, sh):
            return sh + jnp.where((rng >> s) >= _NB, 1, 0)

        sh = lax.fori_loop(0, 31 - 8, shstep, jnp.int32(0))

        def bucket(v):
            return (v - mn) >> sh

        def hzero(i, _):
            hist_v[pl.ds(i * _LANES, _LANES)] = jnp.zeros((_LANES,), jnp.int32)
            return ()

        lax.fori_loop(0, _NB, hzero, (), unroll=8)

        def hfill(i, _):
            v = sbits_v[pl.ds(i * _LANES, _LANES)]
            addr = bucket(v) * _LANES + lane
            cur = plsc.load_gather(hist_v, [addr])
            plsc.store_scatter(hist_v, [addr], cur + 1)
            return ()

        lax.fori_loop(0, nv, hfill, (), unroll=4)

        def scan(i, carry):
            cum, tb, above = carry
            j = _NB - 1 - i
            c = cum + jnp.sum(hist_v[pl.ds(j * _LANES, _LANES)])
            found = (tb < 0) & (c >= K)
            return (c, jnp.where(found, j, tb), jnp.where(found, cum, above))

        _, tb, above = lax.fori_loop(
            0, _NB, scan, (jnp.int32(0), jnp.int32(-1), jnp.int32(0)),
            unroll=4)

        def comp(i, m):
            v = sbits_v[pl.ds(i * _LANES, _LANES)]
            msk = bucket(v) == tb
            plsc.store_compressed(cand_v.at[pl.ds(m, _LANES)], v, mask=msk)
            return m + jnp.sum(jnp.where(msk, 1, 0))

        m = lax.fori_loop(0, nv, comp, jnp.int32(0), unroll=4)
        cand_v[pl.ds(m, _LANES)] = jnp.zeros((_LANES,), jnp.int32)
        nv2 = (m + _LANES - 1) // _LANES

        def count_ge_cand(t):
            def body(i, acc):
                v = cand_v[pl.ds(i * _LANES, _LANES)]
                return acc + jnp.where(v >= t, 1, 0)

            acc = lax.fori_loop(0, nv2, body,
                                jnp.zeros((_LANES,), jnp.int32))
            return jnp.sum(acc)

        Kc = K - above

        def step(k, prefix):
            cand = prefix | (jnp.int32(1) << (30 - k))
            return jnp.where(count_ge_cand(cand) >= Kc, cand, prefix)

        thr = lax.fori_loop(0, 31, step, jnp.int32(0))
        count_gt = above + count_ge_cand(thr + 1)
        count_eq = above + count_ge_cand(thr) - count_gt
        n_eq = K - count_gt

        def emit_fast(i, off):
            v = sbits_v[pl.ds(i * _LANES, _LANES)]
            keep = v >= thr
            plsc.store_compressed(idxbuf_v.at[pl.ds(off, _LANES)],
                                  lane + i * _LANES, mask=keep)
            return off + jnp.sum(jnp.where(keep, 1, 0))

        def emit_slow(i, carry):
            off, eq_seen = carry
            v = sbits_v[pl.ds(i * _LANES, _LANES)]
            gt = v > thr
            eq = v == thr
            eqi = jnp.where(eq, 1, 0)
            excl = plsc.cumsum(eqi) - eqi
            keep = gt | (eq & ((eq_seen + excl) < n_eq))
            plsc.store_compressed(idxbuf_v.at[pl.ds(off, _LANES)],
                                  lane + i * _LANES, mask=keep)
            return (off + jnp.sum(jnp.where(keep, 1, 0)),
                    eq_seen + jnp.sum(eqi))

        def do_fast():
            lax.fori_loop(0, nv, emit_fast, jnp.int32(0), unroll=4)

        def do_slow():
            lax.fori_loop(0, nv, emit_slow, (jnp.int32(0), jnp.int32(0)),
                          unroll=4)

        lax.cond(n_eq == count_eq, do_fast, do_slow)
        pltpu.sync_copy(idxbuf_v.at[pl.ds(0, K)], idx_hbm.at[w])


def _sc_select(score, K):
    B, L = score.shape
    mesh = plsc.VectorSubcoreMesh(core_axis_name="c", subcore_axis_name="s")
    return pl.kernel(
        functools.partial(_select_body, K),
        out_type=jax.ShapeDtypeStruct((B, K), jnp.int32),
        mesh=mesh,
        scratch_types=[
            pltpu.VMEM((L,), jnp.int32),
            pltpu.VMEM((K + _LANES,), jnp.int32),
            pltpu.VMEM((_NB * _LANES,), jnp.int32),
            pltpu.VMEM((L + _LANES,), jnp.int32),
        ],
        compiler_params=pltpu.CompilerParams(needs_layout_passes=False),
    )(score)


_CH = 16


def _gather_body(x_hbm, idx_hbm, out_hbm, idx_v, buf0, buf1, buf2,
                 g0, g1, g2, s0, s1, s2):
    B, L, C = x_hbm.shape
    K = idx_hbm.shape[1]
    per_b = _NW // B
    rows = K // per_b
    nch = rows // _CH
    w = _wid()
    b = w // per_b
    base = (w % per_b) * rows

    pltpu.sync_copy(idx_hbm.at[b, pl.ds(base, rows)], idx_v)

    bufs = (buf0, buf1, buf2)
    gsems = (g0, g1, g2)
    ssems = (s0, s1, s2)

    def gth(j, t):
        return pltpu.make_async_copy(
            x_hbm.at[b].at[idx_v.at[pl.ds(j * _CH, _CH)]], bufs[t], gsems[t])

    def sct(j, t):
        return pltpu.make_async_copy(
            bufs[t], out_hbm.at[b, pl.ds(base + j * _CH, _CH), :], ssems[t])

    gth(0, 0).start()
    gth(1, 1).start()
    gth(0, 0).wait()
    sct(0, 0).start()
    gth(2, 2).start()
    gth(1, 1).wait()
    sct(1, 1).start()
    sct(0, 0).wait()
    gth(3, 0).start()

    def three(k, _):
        j0 = 2 + k * 3
        for t_off in range(3):
            j = j0 + t_off
            t = (2 + t_off) % 3
            gth(j, t).wait()
            sct(j, t).start()
            sct(j - 1, (t + 2) % 3).wait()
            gth(j + 2, (t + 2) % 3).start()
        return ()

    lax.fori_loop(0, (nch - 5) // 3, three, ())
    j = nch - 3
    gth(j, j % 3).wait()
    sct(j, j % 3).start()
    sct(j - 1, (j - 1) % 3).wait()
    gth(j + 2, (j + 2) % 3).start()
    j = nch - 2
    gth(j, j % 3).wait()
    sct(j, j % 3).start()
    sct(j - 1, (j - 1) % 3).wait()
    j = nch - 1
    gth(j, j % 3).wait()
    sct(j, j % 3).start()
    sct(j - 1, (j - 1) % 3).wait()
    sct(j, j % 3).wait()


def _sc_gather(x, idx):
    B, L, C = x.shape
    K = idx.shape[1]
    rows = K // (_NW // B)
    mesh = plsc.VectorSubcoreMesh(core_axis_name="c", subcore_axis_name="s")
    return pl.kernel(
        _gather_body,
        out_type=jax.ShapeDtypeStruct((B, K, C), jnp.float32),
        mesh=mesh,
        scratch_types=[
            pltpu.VMEM((rows,), jnp.int32),
            pltpu.VMEM((_CH, C), jnp.float32),
            pltpu.VMEM((_CH, C), jnp.float32),
            pltpu.VMEM((_CH, C), jnp.float32),
            pltpu.SemaphoreType.DMA,
            pltpu.SemaphoreType.DMA,
            pltpu.SemaphoreType.DMA,
            pltpu.SemaphoreType.DMA,
            pltpu.SemaphoreType.DMA,
            pltpu.SemaphoreType.DMA,
        ],
    )(x, idx)


def kernel(x):
    B, L, C = x.shape
    K = max(1, int(L * (1.0 - _SPARSE_RATIO)))
    score = jnp.sqrt(jnp.sum(x * x, axis=-1))
    indices = _sc_select(lax.bitcast_convert_type(score, jnp.int32), K)
    x_sparse = _sc_gather(x, indices)
    return (x_sparse, indices)


# speedup vs baseline: 2.1865x; 1.0306x over previous
import functools

import jax
import jax.numpy as jnp
from jax import lax
from jax.experimental import pallas as pl
from jax.experimental.pallas import tpu as pltpu
from jax.experimental.pallas import tpu_sc as plsc

_SPARSE_RATIO = 0.5
_NC, _NS, _LANES = 2, 16, 16
_NW = _NC * _NS


def _wid():
    return lax.axis_index("s") * _NC + lax.axis_index("c")


def _select_body(K, score_hbm, idx_hbm, sbits_v, idxbuf_v):
    B, L = score_hbm.shape
    nv = L // _LANES
    w = _wid()

    @pl.when(w < B)
    def _():
        pltpu.sync_copy(score_hbm.at[w], sbits_v)
        lane = lax.iota(jnp.int32, _LANES)

        def count_ge(t):
            def body(i, acc):
                v = sbits_v[pl.ds(i * _LANES, _LANES)]
                return acc + jnp.where(v >= t, 1, 0)

            acc = lax.fori_loop(0, nv, body, jnp.zeros((_LANES,), jnp.int32),
                                unroll=8)
            return jnp.sum(acc)

        def step(k, prefix):
            cand = prefix | (jnp.int32(1) << (30 - k))
            return jnp.where(count_ge(cand) >= K, cand, prefix)

        thr = lax.fori_loop(0, 31, step, jnp.int32(0))
        n_eq = K - count_ge(thr + 1)

        def emit(i, carry):
            off, eq_seen = carry
            v = sbits_v[pl.ds(i * _LANES, _LANES)]
            gt = v > thr
            eq = v == thr
            eqi = jnp.where(eq, 1, 0)
            excl = plsc.cumsum(eqi) - eqi
            keep = gt | (eq & ((eq_seen + excl) < n_eq))
            plsc.store_compressed(idxbuf_v.at[pl.ds(off, _LANES)],
                                  lane + i * _LANES, mask=keep)
            return (off + jnp.sum(jnp.where(keep, 1, 0)),
                    eq_seen + jnp.sum(eqi))

        lax.fori_loop(0, nv, emit, (jnp.int32(0), jnp.int32(0)), unroll=4)
        pltpu.sync_copy(idxbuf_v.at[pl.ds(0, K)], idx_hbm.at[w])


def _sc_select(score, K):
    B, L = score.shape
    mesh = plsc.VectorSubcoreMesh(core_axis_name="c", subcore_axis_name="s")
    return pl.kernel(
        functools.partial(_select_body, K),
        out_type=jax.ShapeDtypeStruct((B, K), jnp.int32),
        mesh=mesh,
        scratch_types=[
            pltpu.VMEM((L,), jnp.int32),
            pltpu.VMEM((K + _LANES,), jnp.int32),
        ],
        compiler_params=pltpu.CompilerParams(needs_layout_passes=False),
    )(score)


_CH = 8


def _gather_body(x_hbm, idx_hbm, out_hbm, idx_v, buf0, buf1, buf2,
                 g0, g1, g2, s0, s1, s2):
    B, L, C = x_hbm.shape
    K = idx_hbm.shape[1]
    per_b = _NW // B
    rows = K // per_b
    nch = rows // _CH
    w = _wid()
    b = w // per_b
    base = (w % per_b) * rows

    pltpu.sync_copy(idx_hbm.at[b, pl.ds(base, rows)], idx_v)

    bufs = (buf0, buf1, buf2)
    gsems = (g0, g1, g2)
    ssems = (s0, s1, s2)

    def gth(j, t):
        return pltpu.make_async_copy(
            x_hbm.at[b].at[idx_v.at[pl.ds(j * _CH, _CH)]], bufs[t], gsems[t])

    def sct(j, t):
        return pltpu.make_async_copy(
            bufs[t], out_hbm.at[b, pl.ds(base + j * _CH, _CH), :], ssems[t])

    gth(0, 0).start()
    gth(1, 1).start()
    gth(0, 0).wait()
    sct(0, 0).start()
    gth(2, 2).start()
    gth(1, 1).wait()
    sct(1, 1).start()
    sct(0, 0).wait()
    gth(3, 0).start()

    def three(k, _):
        j0 = 2 + k * 3
        for t_off in range(3):
            j = j0 + t_off
            t = (2 + t_off) % 3
            gth(j, t).wait()
            sct(j, t).start()
            sct(j - 1, (t + 2) % 3).wait()
            gth(j + 2, (t + 2) % 3).start()
        return ()

    lax.fori_loop(0, (nch - 5) // 3, three, ())
    for j in range(2 + 3 * ((nch - 5) // 3), nch - 3):
        gth(j, j % 3).wait()
        sct(j, j % 3).start()
        sct(j - 1, (j - 1) % 3).wait()
        gth(j + 2, (j + 2) % 3).start()
    j = nch - 3
    gth(j, j % 3).wait()
    sct(j, j % 3).start()
    sct(j - 1, (j - 1) % 3).wait()
    gth(j + 2, (j + 2) % 3).start()
    j = nch - 2
    gth(j, j % 3).wait()
    sct(j, j % 3).start()
    sct(j - 1, (j - 1) % 3).wait()
    j = nch - 1
    gth(j, j % 3).wait()
    sct(j, j % 3).start()
    sct(j - 1, (j - 1) % 3).wait()
    sct(j, j % 3).wait()


def _sc_gather(x, idx):
    B, L, C = x.shape
    K = idx.shape[1]
    rows = K // (_NW // B)
    mesh = plsc.VectorSubcoreMesh(core_axis_name="c", subcore_axis_name="s")
    return pl.kernel(
        _gather_body,
        out_type=jax.ShapeDtypeStruct((B, K, C), jnp.float32),
        mesh=mesh,
        scratch_types=[
            pltpu.VMEM((rows,), jnp.int32),
            pltpu.VMEM((_CH, C), jnp.float32),
            pltpu.VMEM((_CH, C), jnp.float32),
            pltpu.VMEM((_CH, C), jnp.float32),
            pltpu.SemaphoreType.DMA,
            pltpu.SemaphoreType.DMA,
            pltpu.SemaphoreType.DMA,
            pltpu.SemaphoreType.DMA,
            pltpu.SemaphoreType.DMA,
            pltpu.SemaphoreType.DMA,
        ],
    )(x, idx)


def kernel(x):
    B, L, C = x.shape
    K = max(1, int(L * (1.0 - _SPARSE_RATIO)))
    score = jnp.sqrt(jnp.sum(x * x, axis=-1))
    indices = _sc_select(lax.bitcast_convert_type(score, jnp.int32), K)
    x_sparse = _sc_gather(x, indices)
    return (x_sparse, indices)


# speedup vs baseline: 2.1912x; 1.0022x over previous
import functools

import jax
import jax.numpy as jnp
from jax import lax
from jax.experimental import pallas as pl
from jax.experimental.pallas import tpu as pltpu
from jax.experimental.pallas import tpu_sc as plsc

_SPARSE_RATIO = 0.5
_NC, _NS, _LANES = 2, 16, 16
_NW = _NC * _NS


def _wid():
    return lax.axis_index("s") * _NC + lax.axis_index("c")


def _select_body(K, score_hbm, idx_hbm, sbits_v, idxbuf_v):
    B, L = score_hbm.shape
    nv = L // _LANES
    w = _wid()

    @pl.when(w < B)
    def _():
        pltpu.sync_copy(score_hbm.at[w], sbits_v)
        lane = lax.iota(jnp.int32, _LANES)

        def count_ge(t):
            def body(i, acc):
                v = sbits_v[pl.ds(i * _LANES, _LANES)]
                return acc + jnp.where(v >= t, 1, 0)

            acc = lax.fori_loop(0, nv, body, jnp.zeros((_LANES,), jnp.int32),
                                unroll=8)
            return jnp.sum(acc)

        def mm(i, c):
            v = sbits_v[pl.ds(i * _LANES, _LANES)]
            return (jnp.minimum(c[0], v), jnp.maximum(c[1], v))

        mnv, mxv = lax.fori_loop(
            0, nv, mm,
            (jnp.full((_LANES,), jnp.int32(0x7F800000)),
             jnp.zeros((_LANES,), jnp.int32)), unroll=8)
        diff = jnp.min(mnv) ^ jnp.max(mxv)

        def hbstep(s, hb):
            return hb + jnp.where((diff >> s) > 0, 1, 0)

        hb = lax.fori_loop(0, 31, hbstep, jnp.int32(0))
        prefix0 = jnp.max(mxv) & ~((jnp.int32(1) << hb) - 1)

        def step(k, prefix):
            cand = prefix | (jnp.int32(1) << (hb - 1 - k))
            return jnp.where(count_ge(cand) >= K, cand, prefix)

        thr = lax.fori_loop(0, hb, step, prefix0)
        n_eq = K - count_ge(thr + 1)

        def emit(i, carry):
            off, eq_seen = carry
            v = sbits_v[pl.ds(i * _LANES, _LANES)]
            gt = v > thr
            eq = v == thr
            eqi = jnp.where(eq, 1, 0)
            excl = plsc.cumsum(eqi) - eqi
            keep = gt | (eq & ((eq_seen + excl) < n_eq))
            plsc.store_compressed(idxbuf_v.at[pl.ds(off, _LANES)],
                                  lane + i * _LANES, mask=keep)
            return (off + jnp.sum(jnp.where(keep, 1, 0)),
                    eq_seen + jnp.sum(eqi))

        lax.fori_loop(0, nv, emit, (jnp.int32(0), jnp.int32(0)), unroll=4)
        pltpu.sync_copy(idxbuf_v.at[pl.ds(0, K)], idx_hbm.at[w])


def _sc_select(score, K):
    B, L = score.shape
    mesh = plsc.VectorSubcoreMesh(core_axis_name="c", subcore_axis_name="s")
    return pl.kernel(
        functools.partial(_select_body, K),
        out_type=jax.ShapeDtypeStruct((B, K), jnp.int32),
        mesh=mesh,
        scratch_types=[
            pltpu.VMEM((L,), jnp.int32),
            pltpu.VMEM((K + _LANES,), jnp.int32),
        ],
        compiler_params=pltpu.CompilerParams(needs_layout_passes=False),
    )(score)


_CH = 16


def _gather_body(x_hbm, idx_hbm, out_hbm, idx_v, buf0, buf1, buf2,
                 g0, g1, g2, s0, s1, s2):
    B, L, C = x_hbm.shape
    K = idx_hbm.shape[1]
    per_b = _NW // B
    rows = K // per_b
    nch = rows // _CH
    w = _wid()
    b = w // per_b
    base = (w % per_b) * rows

    pltpu.sync_copy(idx_hbm.at[b, pl.ds(base, rows)], idx_v)

    bufs = (buf0, buf1, buf2)
    gsems = (g0, g1, g2)
    ssems = (s0, s1, s2)

    def gth(j, t):
        return pltpu.make_async_copy(
            x_hbm.at[b].at[idx_v.at[pl.ds(j * _CH, _CH)]], bufs[t], gsems[t])

    def sct(j, t):
        return pltpu.make_async_copy(
            bufs[t], out_hbm.at[b, pl.ds(base + j * _CH, _CH), :], ssems[t])

    gth(0, 0).start()
    gth(1, 1).start()
    gth(0, 0).wait()
    sct(0, 0).start()
    gth(2, 2).start()
    gth(1, 1).wait()
    sct(1, 1).start()
    sct(0, 0).wait()
    gth(3, 0).start()

    def three(k, _):
        j0 = 2 + k * 3
        for t_off in range(3):
            j = j0 + t_off
            t = (2 + t_off) % 3
            gth(j, t).wait()
            sct(j, t).start()
            sct(j - 1, (t + 2) % 3).wait()
            gth(j + 2, (t + 2) % 3).start()
        return ()

    lax.fori_loop(0, (nch - 5) // 3, three, ())
    for j in range(2 + 3 * ((nch - 5) // 3), nch - 3):
        gth(j, j % 3).wait()
        sct(j, j % 3).start()
        sct(j - 1, (j - 1) % 3).wait()
        gth(j + 2, (j + 2) % 3).start()
    j = nch - 3
    gth(j, j % 3).wait()
    sct(j, j % 3).start()
    sct(j - 1, (j - 1) % 3).wait()
    gth(j + 2, (j + 2) % 3).start()
    j = nch - 2
    gth(j, j % 3).wait()
    sct(j, j % 3).start()
    sct(j - 1, (j - 1) % 3).wait()
    j = nch - 1
    gth(j, j % 3).wait()
    sct(j, j % 3).start()
    sct(j - 1, (j - 1) % 3).wait()
    sct(j, j % 3).wait()


def _sc_gather(x, idx):
    B, L, C = x.shape
    K = idx.shape[1]
    rows = K // (_NW // B)
    mesh = plsc.VectorSubcoreMesh(core_axis_name="c", subcore_axis_name="s")
    return pl.kernel(
        _gather_body,
        out_type=jax.ShapeDtypeStruct((B, K, C), jnp.float32),
        mesh=mesh,
        scratch_types=[
            pltpu.VMEM((rows,), jnp.int32),
            pltpu.VMEM((_CH, C), jnp.float32),
            pltpu.VMEM((_CH, C), jnp.float32),
            pltpu.VMEM((_CH, C), jnp.float32),
            pltpu.SemaphoreType.DMA,
            pltpu.SemaphoreType.DMA,
            pltpu.SemaphoreType.DMA,
            pltpu.SemaphoreType.DMA,
            pltpu.SemaphoreType.DMA,
            pltpu.SemaphoreType.DMA,
        ],
    )(x, idx)


def kernel(x):
    B, L, C = x.shape
    K = max(1, int(L * (1.0 - _SPARSE_RATIO)))
    score = jnp.sqrt(jnp.sum(x * x, axis=-1))
    indices = _sc_select(lax.bitcast_convert_type(score, jnp.int32), K)
    x_sparse = _sc_gather(x, indices)
    return (x_sparse, indices)
